# baseline (speedup 1.0000x reference)
import functools

import jax
import jax.numpy as jnp
from jax import lax
from jax.experimental import pallas as pl
from jax.experimental.pallas import tpu as pltpu
from jax.experimental.pallas import tpu_sc as plsc

_NUM_CORES = 2
_NUM_SUBCORES = 16
_NUM_WORKERS = _NUM_CORES * _NUM_SUBCORES


@functools.lru_cache(maxsize=None)
def _build(batch, vocab, dim):
    assert batch % (8 * _NUM_WORKERS) == 0
    b_per_w = batch // _NUM_WORKERS
    nchunks = 4
    window = 4
    assert b_per_w % nchunks == 0
    chunk = b_per_w // nchunks
    mesh = plsc.VectorSubcoreMesh(core_axis_name="c", subcore_axis_name="s")

    @functools.partial(
        pl.kernel,
        mesh=mesh,
        out_type=jax.ShapeDtypeStruct((batch, dim), jnp.float32),
        scratch_types=[
            pltpu.VMEM((b_per_w,), jnp.int32),
            [pltpu.VMEM((chunk, dim), jnp.float32) for _ in range(nchunks)],
            [pltpu.SemaphoreType.DMA for _ in range(nchunks)],
            pltpu.SemaphoreType.DMA,
        ],
    )
    def gather_kernel(idx_hbm, table_hbm, out_hbm, idx_v, bufs, gsems, wsem):
        wid = lax.axis_index("s") * _NUM_CORES + lax.axis_index("c")
        base = wid * b_per_w

        def gather(i):
            pltpu.sync_copy(idx_hbm.at[pl.ds(base + i * chunk, chunk)],
                            idx_v.at[pl.ds(i * chunk, chunk)])
            return pltpu.async_copy(
                table_hbm.at[idx_v.at[pl.ds(i * chunk, chunk)]],
                bufs[i], gsems[i])

        gathers = [gather(i) for i in range(window)]
        writes = []
        for i in range(nchunks):
            gathers[i].wait()
            writes.append(pltpu.async_copy(
                bufs[i], out_hbm.at[pl.ds(base + i * chunk, chunk)],
                wsem))
            if i + window < nchunks:
                gathers.append(gather(i + window))
        for w in writes:
            w.wait()

    return gather_kernel


def kernel(asin, embedding_table):
    batch = asin.shape[0]
    vocab, dim = embedding_table.shape
    fn = _build(batch, vocab, dim)
    return fn(asin, embedding_table)


# speedup vs baseline: 1.0090x; 1.0090x over previous
import functools

import jax
import jax.numpy as jnp
from jax import lax
from jax.experimental import pallas as pl
from jax.experimental.pallas import tpu as pltpu
from jax.experimental.pallas import tpu_sc as plsc

_NUM_CORES = 2
_NUM_SUBCORES = 16
_NUM_WORKERS = _NUM_CORES * _NUM_SUBCORES


@functools.lru_cache(maxsize=None)
def _build(batch, vocab, dim):
    assert batch % (8 * _NUM_WORKERS) == 0
    b_per_w = batch // _NUM_WORKERS
    sizes = [b_per_w // 4, b_per_w - b_per_w // 4]
    offs = [0]
    for s in sizes[:-1]:
        offs.append(offs[-1] + s)
    nchunks = len(sizes)
    mesh = plsc.VectorSubcoreMesh(core_axis_name="c", subcore_axis_name="s")

    @functools.partial(
        pl.kernel,
        mesh=mesh,
        out_type=jax.ShapeDtypeStruct((batch, dim), jnp.float32),
        scratch_types=[
            pltpu.VMEM((b_per_w,), jnp.int32),
            [pltpu.VMEM((s, dim), jnp.float32) for s in sizes],
            [pltpu.SemaphoreType.DMA for _ in range(nchunks)],
            pltpu.SemaphoreType.DMA,
        ],
    )
    def gather_kernel(idx_hbm, table_hbm, out_hbm, idx_v, bufs, gsems, wsem):
        wid = lax.axis_index("s") * _NUM_CORES + lax.axis_index("c")
        base = wid * b_per_w

        def gather(i):
            pltpu.sync_copy(idx_hbm.at[pl.ds(base + offs[i], sizes[i])],
                            idx_v.at[pl.ds(offs[i], sizes[i])])
            return pltpu.async_copy(
                table_hbm.at[idx_v.at[pl.ds(offs[i], sizes[i])]],
                bufs[i], gsems[i])

        gathers = [gather(i) for i in range(nchunks)]
        writes = []
        for i in range(nchunks):
            gathers[i].wait()
            writes.append(pltpu.async_copy(
                bufs[i], out_hbm.at[pl.ds(base + offs[i], sizes[i])],
                wsem))
        for w in writes:
            w.wait()

    return gather_kernel


def kernel(asin, embedding_table):
    batch = asin.shape[0]
    vocab, dim = embedding_table.shape
    fn = _build(batch, vocab, dim)
    return fn(asin, embedding_table)


# speedup vs baseline: 1.0202x; 1.0111x over previous
import functools

import jax
import jax.numpy as jnp
from jax import lax
from jax.experimental import pallas as pl
from jax.experimental.pallas import tpu as pltpu
from jax.experimental.pallas import tpu_sc as plsc

_NUM_CORES = 2
_NUM_SUBCORES = 16
_NUM_WORKERS = _NUM_CORES * _NUM_SUBCORES


@functools.lru_cache(maxsize=None)
def _build(batch, vocab, dim):
    assert batch % (8 * _NUM_WORKERS) == 0
    b_per_w = batch // _NUM_WORKERS
    nchunks = 2
    window = 2
    assert b_per_w % nchunks == 0
    chunk = b_per_w // nchunks
    mesh = plsc.VectorSubcoreMesh(core_axis_name="c", subcore_axis_name="s")

    @functools.partial(
        pl.kernel,
        mesh=mesh,
        out_type=jax.ShapeDtypeStruct((batch, dim), jnp.float32),
        scratch_types=[
            pltpu.VMEM((b_per_w,), jnp.int32),
            [pltpu.VMEM((chunk, dim), jnp.float32) for _ in range(nchunks)],
            [pltpu.SemaphoreType.DMA for _ in range(nchunks)],
            pltpu.SemaphoreType.DMA,
        ],
    )
    def gather_kernel(idx_hbm, table_hbm, out_hbm, idx_v, bufs, gsems, wsem):
        wid = lax.axis_index("s") * _NUM_CORES + lax.axis_index("c")
        base = wid * b_per_w

        def gather(i):
            pltpu.sync_copy(idx_hbm.at[pl.ds(base + i * chunk, chunk)],
                            idx_v.at[pl.ds(i * chunk, chunk)])
            return pltpu.async_copy(
                table_hbm.at[idx_v.at[pl.ds(i * chunk, chunk)]],
                bufs[i], gsems[i])

        gathers = [gather(i) for i in range(window)]
        writes = []
        for i in range(nchunks):
            gathers[i].wait()
            writes.append(pltpu.async_copy(
                bufs[i], out_hbm.at[pl.ds(base + i * chunk, chunk)],
                wsem))
            if i + window < nchunks:
                gathers.append(gather(i + window))
        for w in writes:
            w.wait()

    return gather_kernel


def kernel(asin, embedding_table):
    batch = asin.shape[0]
    vocab, dim = embedding_table.shape
    fn = _build(batch, vocab, dim)
    return fn(asin, embedding_table)
